# Initial kernel scaffold; baseline (speedup 1.0000x reference)
import jax
import jax.numpy as jnp
from jax.experimental import pallas as pl


def kernel(nf, ef, u, edge_index, params):
    raise NotImplementedError("write your pallas kernel here")


# speedup vs baseline: 2.2869x; 2.2869x over previous
import functools

import jax
import jax.numpy as jnp
from jax import lax
from jax.experimental import pallas as pl
from jax.experimental.pallas import tpu as pltpu
from jax.experimental.pallas import tpu_sc as plsc

N = 10000
E = 160000
F32 = jnp.float32
EPS = 1e-5


def _dot(a, b):
    return jnp.dot(a, b, preferred_element_type=F32)


ROW = lambda i: (0, 0)


def _colsum_kernel(x_ref, out_ref):
    @pl.when(pl.program_id(0) == 0)
    def _init():
        out_ref[...] = jnp.zeros_like(out_ref)

    out_ref[...] += jnp.sum(x_ref[...], axis=0, keepdims=True)


def _colsum(x, BE=8000):
    d = x.shape[1]
    return pl.pallas_call(
        _colsum_kernel,
        grid=(x.shape[0] // BE,),
        in_specs=[pl.BlockSpec((BE, d), lambda i: (i, 0))],
        out_specs=pl.BlockSpec((1, d), ROW),
        out_shape=jax.ShapeDtypeStruct((1, d), F32),
    )(x)


def _colvar_kernel(x_ref, mu_ref, out_ref):
    @pl.when(pl.program_id(0) == 0)
    def _init():
        out_ref[...] = jnp.zeros_like(out_ref)

    c = x_ref[...] - mu_ref[...]
    out_ref[...] += jnp.sum(c * c, axis=0, keepdims=True)


def _colvarsum(x, mu, BE=8000):
    d = x.shape[1]
    return pl.pallas_call(
        _colvar_kernel,
        grid=(x.shape[0] // BE,),
        in_specs=[pl.BlockSpec((BE, d), lambda i: (i, 0)),
                  pl.BlockSpec((1, d), ROW)],
        out_specs=pl.BlockSpec((1, d), ROW),
        out_shape=jax.ShapeDtypeStruct((1, d), F32),
    )(x, mu)


def _wstats_kernel(x_ref, ds_ref, dd_ref, out_ref):
    x = x_ref[...]
    ds = ds_ref[...]
    dd = dd_ref[...]
    inv_e = 1.0 / E
    s1s = jnp.sum(ds * x, axis=0, keepdims=True)
    mus = s1s * inv_e
    cs = x - mus
    vs = jnp.sum(ds * cs * cs, axis=0, keepdims=True)
    s1d = jnp.sum(dd * x, axis=0, keepdims=True)
    mud = s1d * inv_e
    cd = x - mud
    vd = jnp.sum(dd * cd * cd, axis=0, keepdims=True)
    out_ref[...] = jnp.concatenate([mus, vs, mud, vd], axis=0)


def _wstats(x, ds, dd):
    dn = x.shape[1]
    args = (x, ds, dd)
    return pl.pallas_call(
        _wstats_kernel,
        in_specs=[pl.BlockSpec((N, dn), lambda: (0, 0)),
                  pl.BlockSpec((N, 1), lambda: (0, 0)),
                  pl.BlockSpec((N, 1), lambda: (0, 0))],
        out_specs=pl.BlockSpec((4, dn), lambda: (0, 0)),
        out_shape=jax.ShapeDtypeStruct((4, dn), F32),
    )(*args)


def _edge_kernel(residual, dn,
                 ef_ref, sf_ref, df_ref,
                 ae_ref, as_ref, ad_ref, mue_ref, mus_ref, mud_ref,
                 be_ref, bs_ref, bd_ref, bu_ref,
                 w1e_ref, w1s_ref, w1d_ref, w1u_ref, b1_ref,
                 w2_ref, b2_ref, w3_ref, b3_ref,
                 out_ref, pre_ref, st_ref, stp_ref):
    const = _dot(bu_ref[...], w1u_ref[...]) + b1_ref[...]
    ef = ef_ref[...]
    sf = sf_ref[...][:, :dn]
    df = df_ref[...][:, :dn]
    h = _dot((ef - mue_ref[...]) * ae_ref[...] + be_ref[...], w1e_ref[...])
    h = h + _dot((sf - mus_ref[...]) * as_ref[...] + bs_ref[...],
                 w1s_ref[...])
    h = h + _dot((df - mud_ref[...]) * ad_ref[...] + bd_ref[...],
                 w1d_ref[...])
    h = jnp.maximum(h + const, 0.0)
    h = jnp.maximum(_dot(h, w2_ref[...]) + b2_ref[...], 0.0)
    e3 = jnp.maximum(_dot(h, w3_ref[...]) + b3_ref[...], 0.0)
    pre_ref[...] = jnp.pad(e3, ((0, 0), (0, 96)))
    if residual:
        e3 = e3 + ef
    out_ref[...] = e3

    @pl.when(pl.program_id(0) == 0)
    def _init():
        st_ref[...] = jnp.zeros_like(st_ref)
        stp_ref[...] = jnp.zeros_like(stp_ref)

    st_ref[...] += jnp.sum(e3, axis=0, keepdims=True)
    stp_ref[...] += jnp.sum(pre_ref[...][:, :32], axis=0, keepdims=True)


def _scale(varsum, denom, gg):
    return (gg * lax.rsqrt(varsum * (1.0 / denom) + EPS))[None, :]


def _edge_mlp(lp, ef, sfeat, dfeat, mu_ef, efV, nfst, u_norm, residual):
    de, dn = ef.shape[1], nfst.shape[1]
    dg = lp["em"]["Ws"][0].shape[0] - de - 2 * dn
    g, b = lp["em"]["gamma"], lp["em"]["beta"]

    a_e = _scale(efV[0], E, g[:de])
    a_s = _scale(nfst[1], E, g[de:de + dn])
    a_d = _scale(nfst[3], E, g[de + dn:de + 2 * dn])
    mu_s = nfst[0][None, :]
    mu_d = nfst[2][None, :]
    b_e = b[None, :de]
    b_s = b[None, de:de + dn]
    b_d = b[None, de + dn:de + 2 * dn]
    b_u = u_norm

    W1 = lp["em"]["Ws"][0]
    W1e, W1s = W1[:de], W1[de:de + dn]
    W1d, W1u = W1[de + dn:de + 2 * dn], W1[de + 2 * dn:]
    b1 = lp["em"]["bs"][0][None, :]
    W2, b2 = lp["em"]["Ws"][1], lp["em"]["bs"][1][None, :]
    W3, b3 = lp["em"]["Ws"][2], lp["em"]["bs"][2][None, :]

    BE = 2000
    e_new, e_pre, st, stp = pl.pallas_call(
        functools.partial(_edge_kernel, residual, dn),
        grid=(E // BE,),
        in_specs=[
            pl.BlockSpec((BE, de), lambda i: (i, 0)),
            pl.BlockSpec((BE, 128), lambda i: (i, 0)),
            pl.BlockSpec((BE, 128), lambda i: (i, 0)),
            pl.BlockSpec((1, de), ROW), pl.BlockSpec((1, dn), ROW),
            pl.BlockSpec((1, dn), ROW), pl.BlockSpec((1, de), ROW),
            pl.BlockSpec((1, dn), ROW), pl.BlockSpec((1, dn), ROW),
            pl.BlockSpec((1, de), ROW), pl.BlockSpec((1, dn), ROW),
            pl.BlockSpec((1, dn), ROW), pl.BlockSpec((1, dg), ROW),
            pl.BlockSpec((de, 256), ROW), pl.BlockSpec((dn, 256), ROW),
            pl.BlockSpec((dn, 256), ROW), pl.BlockSpec((dg, 256), ROW),
            pl.BlockSpec((1, 256), ROW),
            pl.BlockSpec((256, 128), ROW), pl.BlockSpec((1, 128), ROW),
            pl.BlockSpec((128, 32), ROW), pl.BlockSpec((1, 32), ROW),
        ],
        out_specs=[
            pl.BlockSpec((BE, 32), lambda i: (i, 0)),
            pl.BlockSpec((BE, 128), lambda i: (i, 0)),
            pl.BlockSpec((1, 32), ROW),
            pl.BlockSpec((1, 32), ROW),
        ],
        out_shape=[
            jax.ShapeDtypeStruct((E, 32), F32),
            jax.ShapeDtypeStruct((E, 128), F32),
            jax.ShapeDtypeStruct((1, 32), F32),
            jax.ShapeDtypeStruct((1, 32), F32),
        ],
    )(ef, sfeat, dfeat, a_e, a_s, a_d, mu_ef, mu_s, mu_d, b_e, b_s, b_d, b_u,
      W1e, W1s, W1d, W1u, b1, W2, b2, W3, b3)
    return e_new, e_pre, st, stp


def _nstats_kernel(nf_ref, p0_ref, p1_ref, deg_ref, nfst_ref, agst_ref):
    inv_n = 1.0 / N
    nf = nf_ref[...]
    agg = (p0_ref[...] + p1_ref[...]) / jnp.maximum(deg_ref[...], 1.0)
    mun = jnp.sum(nf, axis=0, keepdims=True) * inv_n
    cn = nf - mun
    vn = jnp.sum(cn * cn, axis=0, keepdims=True)
    mua = jnp.sum(agg, axis=0, keepdims=True) * inv_n
    ca = agg - mua
    va = jnp.sum(ca * ca, axis=0, keepdims=True)
    nfst_ref[...] = jnp.concatenate([mun, vn], axis=0)
    agst_ref[...] = jnp.concatenate([mua, va], axis=0)


def _node_stats(nf, p0, p1, deg_dst):
    dn = nf.shape[1]
    return pl.pallas_call(
        _nstats_kernel,
        in_specs=[pl.BlockSpec((N, dn), lambda: (0, 0)),
                  pl.BlockSpec((N, 32), lambda: (0, 0)),
                  pl.BlockSpec((N, 32), lambda: (0, 0)),
                  pl.BlockSpec((N, 1), lambda: (0, 0))],
        out_specs=[pl.BlockSpec((2, dn), lambda: (0, 0)),
                   pl.BlockSpec((2, 32), lambda: (0, 0))],
        out_shape=[jax.ShapeDtypeStruct((2, dn), F32),
                   jax.ShapeDtypeStruct((2, 32), F32)],
    )(nf, p0, p1, deg_dst)


def _node_body_kernel(residual,
                      nf_ref, p0_ref, p1_ref, deg_ref,
                      an_ref, mun_ref, bn_ref, aa_ref, mua_ref, ba_ref, bu_ref,
                      w1n_ref, w1a_ref, w1u_ref, b1_ref,
                      w2_ref, b2_ref, w3_ref, b3_ref,
                      nout_ref, mn_ref):
    const = _dot(bu_ref[...], w1u_ref[...]) + b1_ref[...]
    nf = nf_ref[...]
    agg = (p0_ref[...] + p1_ref[...]) / jnp.maximum(deg_ref[...], 1.0)
    h = (_dot((nf - mun_ref[...]) * an_ref[...] + bn_ref[...], w1n_ref[...])
         + _dot((agg - mua_ref[...]) * aa_ref[...] + ba_ref[...], w1a_ref[...]))
    h = jnp.maximum(h + const, 0.0)
    h = jnp.maximum(_dot(h, w2_ref[...]) + b2_ref[...], 0.0)
    n3 = jnp.maximum(_dot(h, w3_ref[...]) + b3_ref[...], 0.0)

    @pl.when(pl.program_id(0) == 0)
    def _init():
        mn_ref[...] = jnp.zeros_like(mn_ref)

    mn_ref[...] += jnp.sum(n3, axis=0, keepdims=True) * (1.0 / N)
    if residual:
        n3 = n3 + nf
    nout_ref[...] = n3


def _glob_kernel(residual,
                 u_ref, me_ref, mn_ref,
                 g1u_ref, g1e_ref, g1n_ref, gb1_ref,
                 g2_ref, gb2_ref, g3_ref, gb3_ref, uout_ref):
    u = u_ref[...]
    hg = (_dot(u, g1u_ref[...]) + _dot(me_ref[...], g1e_ref[...])
          + _dot(mn_ref[...], g1n_ref[...]) + gb1_ref[...])
    hg = jnp.maximum(hg, 0.0)
    hg = jnp.maximum(_dot(hg, g2_ref[...]) + gb2_ref[...], 0.0)
    u3 = jnp.maximum(_dot(hg, g3_ref[...]) + gb3_ref[...], 0.0)
    if residual:
        u3 = u3 + u
    uout_ref[...] = u3


def _node_mlp(lp, nf, p0, p1, deg_dst, u, mean_e, u_norm, residual):
    dn = nf.shape[1]
    dg = u.shape[1]
    g, b = lp["nm"]["gamma"], lp["nm"]["beta"]
    W1 = lp["nm"]["Ws"][0]
    W1n, W1a, W1u = W1[:dn], W1[dn:dn + 32], W1[dn + 32:]
    G1 = lp["gm"]["Ws"][0]
    G1u, G1e, G1n = G1[:dg], G1[dg:dg + 32], G1[dg + 32:]

    nfst, agst = _node_stats(nf, p0, p1, deg_dst)
    a_n = _scale(nfst[1], N, g[:dn])
    a_a = _scale(agst[1], N, g[dn:dn + 32])
    mu_n = nfst[0][None, :]
    mu_a = agst[0][None, :]
    b_n = b[None, :dn]
    b_a = b[None, dn:dn + 32]
    b_u = u_norm

    BN = 2000
    blk = lambda w: pl.BlockSpec((BN, w), lambda i: (i, 0))
    n_new, mean_n = pl.pallas_call(
        functools.partial(_node_body_kernel, residual),
        grid=(N // BN,),
        in_specs=[
            blk(dn), blk(32), blk(32), blk(1),
            pl.BlockSpec((1, dn), ROW), pl.BlockSpec((1, dn), ROW),
            pl.BlockSpec((1, dn), ROW),
            pl.BlockSpec((1, 32), ROW), pl.BlockSpec((1, 32), ROW),
            pl.BlockSpec((1, 32), ROW), pl.BlockSpec((1, dg), ROW),
            pl.BlockSpec((dn, 256), ROW), pl.BlockSpec((32, 256), ROW),
            pl.BlockSpec((dg, 256), ROW), pl.BlockSpec((1, 256), ROW),
            pl.BlockSpec((256, 128), ROW), pl.BlockSpec((1, 128), ROW),
            pl.BlockSpec((128, 32), ROW), pl.BlockSpec((1, 32), ROW),
        ],
        out_specs=[
            pl.BlockSpec((BN, 32), lambda i: (i, 0)),
            pl.BlockSpec((1, 32), ROW),
        ],
        out_shape=[
            jax.ShapeDtypeStruct((N, 32), F32),
            jax.ShapeDtypeStruct((1, 32), F32),
        ],
    )(nf, p0, p1, deg_dst,
      a_n, mu_n, b_n, a_a, mu_a, b_a, b_u,
      W1n, W1a, W1u, lp["nm"]["bs"][0][None, :],
      lp["nm"]["Ws"][1], lp["nm"]["bs"][1][None, :],
      lp["nm"]["Ws"][2], lp["nm"]["bs"][2][None, :])

    full = lambda a: pl.BlockSpec(a.shape, lambda: tuple(0 for _ in a.shape))
    gargs = (u, mean_e, mean_n,
             G1u, G1e, G1n, lp["gm"]["bs"][0][None, :],
             lp["gm"]["Ws"][1], lp["gm"]["bs"][1][None, :],
             lp["gm"]["Ws"][2], lp["gm"]["bs"][2][None, :])
    u_new = pl.pallas_call(
        functools.partial(_glob_kernel, residual),
        in_specs=[full(a) for a in gargs],
        out_specs=pl.BlockSpec((1, 32), lambda: (0, 0)),
        out_shape=jax.ShapeDtypeStruct((1, 32), F32),
    )(*gargs)
    return n_new, u_new


def _reg_kernel(nf_ref, w1_ref, b1_ref, w2_ref, b2_ref, w3_ref, b3_ref,
                out_ref):
    h = jnp.maximum(_dot(nf_ref[...], w1_ref[...]) + b1_ref[...], 0.0)
    h = jnp.maximum(_dot(h, w2_ref[...]) + b2_ref[...], 0.0)
    p = _dot(h, w3_ref[...]) + b3_ref[...]
    out_ref[...] = jnp.clip(p, 0.0, 1.0)


def _reg_mlp(rp, nf):
    BN = 2000
    args = (nf, rp["Ws"][0], rp["bs"][0][None, :],
            rp["Ws"][1], rp["bs"][1][None, :],
            rp["Ws"][2], rp["bs"][2][None, :])
    return pl.pallas_call(
        _reg_kernel,
        grid=(N // BN,),
        in_specs=[
            pl.BlockSpec((BN, 32), lambda i: (i, 0)),
            pl.BlockSpec((32, 256), ROW), pl.BlockSpec((1, 256), ROW),
            pl.BlockSpec((256, 128), ROW), pl.BlockSpec((1, 128), ROW),
            pl.BlockSpec((128, 1), ROW), pl.BlockSpec((1, 1), ROW),
        ],
        out_specs=pl.BlockSpec((BN, 1), lambda i: (i, 0)),
        out_shape=jax.ShapeDtypeStruct((N, 1), F32),
    )(*args)


_CH = 128
_NCH = E // _CH
_NW = 32
_PER_W = -(-_NCH // _NW)
_MESH = dict(core_axis_name="c", subcore_axis_name="s")


def _sc_gather(nf, src, dst):
    if nf.shape[1] < 128:
        nf = jnp.pad(nf, ((0, 0), (0, 128 - nf.shape[1])))
    dn = nf.shape[1]

    @functools.partial(
        pl.kernel,
        out_type=[jax.ShapeDtypeStruct((E, dn), F32),
                  jax.ShapeDtypeStruct((E, dn), F32)],
        mesh=plsc.VectorSubcoreMesh(**_MESH),
        scratch_types=[
            pltpu.VMEM((_CH,), jnp.int32),
            pltpu.VMEM((_CH,), jnp.int32),
            pltpu.VMEM((_CH, dn), F32),
            pltpu.VMEM((_CH, dn), F32),
            pltpu.SemaphoreType.DMA,
            pltpu.SemaphoreType.DMA,
        ],
    )
    def k(nf_hbm, src_hbm, dst_hbm, os_hbm, od_hbm,
          si_v, di_v, sr_v, dr_v, s_sem, d_sem):
        wid = lax.axis_index("s") * 2 + lax.axis_index("c")

        def body(j, carry):
            c = wid + _NW * j

            @pl.when(c < _NCH)
            def _():
                base = c * _CH
                pltpu.sync_copy(src_hbm.at[pl.ds(base, _CH)], si_v)
                pltpu.sync_copy(dst_hbm.at[pl.ds(base, _CH)], di_v)
                cs = pltpu.async_copy(nf_hbm.at[si_v], sr_v, s_sem)
                cd = pltpu.async_copy(nf_hbm.at[di_v], dr_v, d_sem)
                cs.wait()
                cd.wait()
                pltpu.sync_copy(sr_v, os_hbm.at[pl.ds(base, _CH)])
                pltpu.sync_copy(dr_v, od_hbm.at[pl.ds(base, _CH)])
            return carry

        lax.fori_loop(0, _PER_W, body, 0)

    return k(nf, src, dst)


def _sc_scatter(vals, idx, width):
    zeros = jnp.zeros((N, width), F32)

    @functools.partial(
        pl.kernel,
        out_type=jax.ShapeDtypeStruct((2, N, width), F32),
        mesh=plsc.VectorSubcoreMesh(**_MESH),
        scratch_types=[
            pltpu.VMEM((_CH,), jnp.int32),
            pltpu.VMEM((_CH, width), F32),
            pltpu.VMEM_SHARED((N, width), F32),
        ],
    )
    def k(vals_hbm, idx_hbm, z_hbm, out_hbm, i_v, r_v, shared):
        cid = lax.axis_index("c")
        sid = lax.axis_index("s")
        wid = sid * 2 + cid

        @pl.when(sid == 0)
        def _init():
            pltpu.sync_copy(z_hbm, shared)

        plsc.subcore_barrier()

        def body(j, carry):
            c = wid + _NW * j

            @pl.when(c < _NCH)
            def _():
                base = c * _CH
                pltpu.sync_copy(idx_hbm.at[pl.ds(base, _CH)], i_v)
                pltpu.sync_copy(vals_hbm.at[pl.ds(base, _CH)], r_v)
                pltpu.sync_copy(r_v, shared.at[i_v], add=True)
            return carry

        lax.fori_loop(0, _PER_W, body, 0)
        plsc.subcore_barrier()

        @pl.when(sid == 0)
        def _out():
            pltpu.sync_copy(shared, out_hbm.at[cid])

    return k(vals, idx, zeros)


def _gather(nf, src, dst):
    return _sc_gather(nf, src, dst)


def _scatter(e_val, dst):
    p = _sc_scatter(e_val, dst, 128)
    return p[0, :, :32], p[1, :, :32]


def _degrees(src, dst):
    ones = jnp.ones((E, 128), F32)
    ps = _sc_scatter(ones, src, 128)
    pd = _sc_scatter(ones, dst, 128)
    return (ps[0, :, 0] + ps[1, :, 0]), (pd[0, :, 0] + pd[1, :, 0])


def kernel(nf, ef, u, edge_index, params):
    src, dst = edge_index[0], edge_index[1]
    deg_src, deg_dst = _degrees(src, dst)
    dsrc_c = deg_src[:, None]
    ddst_c = deg_dst[:, None]

    efS1 = _colsum(ef)
    mu_ef = efS1 * (1.0 / E)
    efV = _colvarsum(ef, mu_ef)
    nfst4 = _wstats(nf, dsrc_c, ddst_c)

    n_layers = len(params["layers"])
    for i, lp in enumerate(params["layers"]):
        residual = i >= 1
        dn = nf.shape[1]
        dg = u.shape[1]
        de = ef.shape[1]

        def u_norm_row(n_rows, gg, bb):
            bc = jnp.broadcast_to(u, (n_rows, dg))
            mu_u = jnp.mean(bc, axis=0)
            var_u = jnp.var(bc, axis=0)
            return ((u[0] - mu_u) / jnp.sqrt(var_u + EPS) * gg + bb)[None, :]

        un_e = u_norm_row(E, lp["em"]["gamma"][de + 2 * dn:],
                          lp["em"]["beta"][de + 2 * dn:])
        un_n = u_norm_row(N, lp["nm"]["gamma"][dn + 32:],
                          lp["nm"]["beta"][dn + 32:])

        sfeat, dfeat = _gather(nf, src, dst)
        e_new, e_pre, esum, epsum = _edge_mlp(
            lp, ef, sfeat, dfeat, mu_ef, efV, nfst4, un_e, residual)
        p0, p1 = _scatter(e_pre, dst)
        mean_e = epsum * (1.0 / E)
        n_new, u_new = _node_mlp(lp, nf, p0, p1, ddst_c, u, mean_e, un_n,
                                 residual)
        if i + 1 < n_layers:
            mu_ef = esum * (1.0 / E)
            efV = _colvarsum(e_new, mu_ef)
            nfst4 = _wstats(n_new, dsrc_c, ddst_c)
        nf, ef, u = n_new, e_new, u_new

    return _reg_mlp(params["reg"], nf)
